# Initial kernel scaffold; baseline (speedup 1.0000x reference)
import jax
import jax.numpy as jnp
from jax.experimental import pallas as pl


def kernel(x, edge_index_k_hops, Wg, att_src, att_dst, bg, Wd, bd):
    raise NotImplementedError("write your pallas kernel here")


# speedup vs baseline: 41.2870x; 41.2870x over previous
import functools

import jax
import jax.numpy as jnp
from jax import lax
from jax.experimental import pallas as pl
from jax.experimental.pallas import tpu as pltpu
from jax.experimental.pallas import tpu_sc as plsc

N = 10000
E = 320000
HOPS = 3
HEADS = 8
HID = 128
HD = HID // HEADS
OUT = 128
DECAY = (0.6, 0.3, 0.1)

NWORKERS = 32
PER_TILE = E // NWORKERS
K = 80
NCHUNK = PER_TILE // K
ROWS_A = 640
ROWS_B = N - 15 * ROWS_A
RB = 1000
NB = N // RB


def _leaky(v, slope):
    return jnp.where(v > 0, v, v * slope)


def _prep_body(x_ref, wg_ref, asrcm_ref, adstm_ref,
               h_ref, as_ref, ad_ref, mxs_ref, mxd_ref):
    j = pl.program_id(1)
    h = jnp.dot(x_ref[...], wg_ref[0], preferred_element_type=jnp.float32)
    h_ref[0] = h
    asv = jnp.dot(h, asrcm_ref[0], preferred_element_type=jnp.float32)
    adv = jnp.dot(h, adstm_ref[0], preferred_element_type=jnp.float32)
    as_ref[0] = asv
    ad_ref[0] = adv
    bs = jnp.max(asv, axis=0)[None, None, :]
    bd = jnp.max(adv, axis=0)[None, None, :]

    @pl.when(j == 0)
    def _():
        mxs_ref[...] = bs
        mxd_ref[...] = bd

    @pl.when(j > 0)
    def _():
        mxs_ref[...] = jnp.maximum(mxs_ref[...], bs)
        mxd_ref[...] = jnp.maximum(mxd_ref[...], bd)


def _prep(x, Wg, Asrc, Adst):
    return pl.pallas_call(
        _prep_body,
        grid=(HOPS, NB),
        in_specs=[
            pl.BlockSpec((RB, HID), lambda k, j: (j, 0)),
            pl.BlockSpec((1, HID, HID), lambda k, j: (k, 0, 0)),
            pl.BlockSpec((1, HID, HEADS), lambda k, j: (k, 0, 0)),
            pl.BlockSpec((1, HID, HEADS), lambda k, j: (k, 0, 0)),
        ],
        out_specs=[
            pl.BlockSpec((1, RB, HID), lambda k, j: (k, j, 0)),
            pl.BlockSpec((1, RB, HEADS), lambda k, j: (k, j, 0)),
            pl.BlockSpec((1, RB, HEADS), lambda k, j: (k, j, 0)),
            pl.BlockSpec((1, 1, HEADS), lambda k, j: (k, 0, 0)),
            pl.BlockSpec((1, 1, HEADS), lambda k, j: (k, 0, 0)),
        ],
        out_shape=[
            jax.ShapeDtypeStruct((HOPS, N, HID), jnp.float32),
            jax.ShapeDtypeStruct((HOPS, N, HEADS), jnp.float32),
            jax.ShapeDtypeStruct((HOPS, N, HEADS), jnp.float32),
            jax.ShapeDtypeStruct((HOPS, 1, HEADS), jnp.float32),
            jax.ShapeDtypeStruct((HOPS, 1, HEADS), jnp.float32),
        ],
    )(x, Wg, Asrc, Adst)


def _sc_edges(edges, h_flat, alphaA, barr):
    mesh = plsc.VectorSubcoreMesh(core_axis_name="c", subcore_axis_name="s")

    @functools.partial(
        pl.kernel,
        mesh=mesh,
        out_type=[
            jax.ShapeDtypeStruct((HOPS * 2, N, HID), jnp.float32),
            jax.ShapeDtypeStruct((HOPS * E * 16,), jnp.float32),
        ],
        scratch_types=[
            pltpu.VMEM((K,), jnp.int32),
            pltpu.VMEM((K,), jnp.int32),
            pltpu.VMEM((K,), jnp.int32),
            pltpu.VMEM((K, HID), jnp.float32),
            pltpu.VMEM((K, 16), jnp.float32),
            pltpu.VMEM((K * 16,), jnp.float32),
            pltpu.VMEM((K, HID), jnp.float32),
            pltpu.VMEM((16,), jnp.float32),
            pltpu.VMEM_SHARED((N, HID), jnp.float32),
            pltpu.SemaphoreType.DMA,
            pltpu.SemaphoreType.DMA,
        ],
    )
    def body(edges_hbm, h_hbm, aA_hbm, b_hbm,
             accp_hbm, ex_hbm,
             srcb, dstb, dstk, gdbuf, exbuf, packed, rows, bbuf,
             acc_sh, sem1, sem3):
        c = lax.axis_index("c")
        s = lax.axis_index("s")
        wid = s * 2 + c
        zero16 = jnp.zeros((16,), jnp.float32)

        for k in range(HOPS):
            kc = 2 * k + c

            def zrow(e, _):
                for j in range(HEADS):
                    rows[e, pl.ds(16 * j, 16)] = zero16
                exbuf[e, :] = zero16
                return 0

            lax.fori_loop(0, K, zrow, 0)

            @pl.when(s < 15)
            def _():
                for i in range(ROWS_A // K):
                    pltpu.sync_copy(rows, acc_sh.at[pl.ds(s * ROWS_A + K * i, K)])

            @pl.when(s == 15)
            def _():
                for i in range(ROWS_B // K):
                    pltpu.sync_copy(rows, acc_sh.at[pl.ds(15 * ROWS_A + K * i, K)])

            pltpu.sync_copy(b_hbm.at[pl.ds(16 * k, 16)], bbuf)
            plsc.subcore_barrier()

            def chunk(cc, _):
                base = wid * PER_TILE + cc * K
                soff = k * 2 * E + base
                pltpu.sync_copy(edges_hbm.at[pl.ds(soff, K)], srcb)
                pltpu.sync_copy(edges_hbm.at[pl.ds(soff + E, K)], dstb)
                for i in range(K // 16):
                    sl = pl.ds(16 * i, 16)
                    srcb[sl] = srcb[sl] + k * N
                    dstk[sl] = dstb[sl] + k * N
                cp_h = pltpu.async_copy(h_hbm.at[srcb], rows, sem3)
                cp_d = pltpu.async_copy(aA_hbm.at[dstk], gdbuf, sem1)
                rot = ((lax.iota(jnp.int32, 16) + 8) & 15)[:, None]
                dnums = lax.GatherDimensionNumbers(
                    offset_dims=(), collapsed_slice_dims=(0,),
                    start_index_map=(0,))
                cp_d.wait()

                def rotloop(e, _):
                    exbuf[e, :] = lax.gather(
                        gdbuf[e, pl.ds(0, 16)], rot, dimension_numbers=dnums,
                        slice_sizes=(1,),
                        mode=lax.GatherScatterMode.PROMISE_IN_BOUNDS)
                    return 0

                lax.fori_loop(0, K, rotloop, 0)
                cp_s = pltpu.async_copy(aA_hbm.at[srcb], gdbuf, sem1)
                cp_s.wait()
                bv = bbuf[:]

                def exloop(e, _):
                    sv = gdbuf[e, pl.ds(0, 16)] + exbuf[e, :]
                    exbuf[e, :] = jnp.exp(_leaky(sv, 0.2) - bv)
                    return 0

                lax.fori_loop(0, K, exloop, 0)
                cp_h.wait()

                def sloop(e, _):
                    ev = exbuf[e, :]
                    for j in range(HEADS):
                        sl = pl.ds(16 * j, 16)
                        rows[e, sl] = rows[e, sl] * ev[j]
                    return 0

                lax.fori_loop(0, K, sloop, 0)
                pltpu.sync_copy(rows, acc_sh.at[dstb], add=True)

                def ploop(e, _):
                    packed[pl.ds(16 * e, 16)] = exbuf[e, :]
                    return 0

                lax.fori_loop(0, K, ploop, 0)
                pltpu.sync_copy(packed,
                                ex_hbm.at[pl.ds((k * E + base) * 16, K * 16)])
                return 0

            lax.fori_loop(0, NCHUNK, chunk, 0)
            plsc.subcore_barrier()

            @pl.when(s < 15)
            def _():
                pltpu.sync_copy(acc_sh.at[pl.ds(s * ROWS_A, ROWS_A)],
                                accp_hbm.at[kc, pl.ds(s * ROWS_A, ROWS_A)])

            @pl.when(s == 15)
            def _():
                pltpu.sync_copy(acc_sh.at[pl.ds(15 * ROWS_A, ROWS_B)],
                                accp_hbm.at[kc, pl.ds(15 * ROWS_A, ROWS_B)])

            plsc.subcore_barrier()

    return body(edges, h_flat, alphaA, barr)


def _sc_den(edges, exflat):
    mesh = plsc.VectorSubcoreMesh(core_axis_name="c", subcore_axis_name="s")

    @functools.partial(
        pl.kernel,
        mesh=mesh,
        out_type=[
            jax.ShapeDtypeStruct((HOPS * 2, N, HID), jnp.float32),
        ],
        scratch_types=[
            pltpu.VMEM((K,), jnp.int32),
            pltpu.VMEM((K * 16,), jnp.float32),
            pltpu.VMEM((K, HID), jnp.float32),
            pltpu.VMEM_SHARED((N, HID), jnp.float32),
        ],
    )
    def body(edges_hbm, ex_hbm, denp_hbm,
             dstb, packed, rows, den_sh):
        c = lax.axis_index("c")
        s = lax.axis_index("s")
        wid = s * 2 + c
        zero16 = jnp.zeros((16,), jnp.float32)

        for k in range(HOPS):
            kc = 2 * k + c

            def zrow(e, _):
                for j in range(HEADS):
                    rows[e, pl.ds(16 * j, 16)] = zero16
                return 0

            lax.fori_loop(0, K, zrow, 0)

            @pl.when(s < 15)
            def _():
                for i in range(ROWS_A // K):
                    pltpu.sync_copy(rows, den_sh.at[pl.ds(s * ROWS_A + K * i, K)])

            @pl.when(s == 15)
            def _():
                for i in range(ROWS_B // K):
                    pltpu.sync_copy(rows, den_sh.at[pl.ds(15 * ROWS_A + K * i, K)])

            plsc.subcore_barrier()

            def chunk(cc, _):
                base = wid * PER_TILE + cc * K
                soff = k * 2 * E + base
                pltpu.sync_copy(edges_hbm.at[pl.ds(soff + E, K)], dstb)
                pltpu.sync_copy(ex_hbm.at[pl.ds((k * E + base) * 16, K * 16)],
                                packed)

                def uloop(e, _):
                    rows[e, pl.ds(0, 16)] = packed[pl.ds(16 * e, 16)]
                    return 0

                lax.fori_loop(0, K, uloop, 0)
                pltpu.sync_copy(rows, den_sh.at[dstb], add=True)
                return 0

            lax.fori_loop(0, NCHUNK, chunk, 0)
            plsc.subcore_barrier()

            @pl.when(s < 15)
            def _():
                pltpu.sync_copy(den_sh.at[pl.ds(s * ROWS_A, ROWS_A)],
                                denp_hbm.at[kc, pl.ds(s * ROWS_A, ROWS_A)])

            @pl.when(s == 15)
            def _():
                pltpu.sync_copy(den_sh.at[pl.ds(15 * ROWS_A, ROWS_B)],
                                denp_hbm.at[kc, pl.ds(15 * ROWS_A, ROWS_B)])

            plsc.subcore_barrier()

    return body(edges, exflat)


def _decode_body(h_ref, as_ref, ad_ref, b_ref, accp_ref, denp_ref,
                 bg_ref, wd_ref, bd_ref, s_ref, out_ref):
    acc = jnp.zeros((RB, OUT), jnp.float32)
    S = s_ref[...]
    for k in range(HOPS):
        exs = jnp.exp(_leaky(as_ref[k] + ad_ref[k], 0.2) - b_ref[k][None, :])
        den = denp_ref[2 * k][:, :HEADS] + denp_ref[2 * k + 1][:, :HEADS] + exs
        exs128 = jnp.dot(exs, S, preferred_element_type=jnp.float32)
        rcp128 = jnp.dot(1.0 / den, S, preferred_element_type=jnp.float32)
        num = accp_ref[2 * k] + accp_ref[2 * k + 1] + h_ref[k] * exs128
        gat = num * rcp128 + bg_ref[k][None, :]
        xk = jnp.dot(gat, wd_ref[k], preferred_element_type=jnp.float32)
        xk = _leaky(xk + bd_ref[k][None, :], 0.01)
        acc = acc + DECAY[k] * xk
    out_ref[...] = acc


def _decode(h3, asrc3, adst3, b3, accp, denp, bg, Wd, bd, S):
    return pl.pallas_call(
        _decode_body,
        grid=(NB,),
        in_specs=[
            pl.BlockSpec((HOPS, RB, HID), lambda j: (0, j, 0)),
            pl.BlockSpec((HOPS, RB, HEADS), lambda j: (0, j, 0)),
            pl.BlockSpec((HOPS, RB, HEADS), lambda j: (0, j, 0)),
            pl.BlockSpec((HOPS, HEADS), lambda j: (0, 0)),
            pl.BlockSpec((HOPS * 2, RB, HID), lambda j: (0, j, 0)),
            pl.BlockSpec((HOPS * 2, RB, HID), lambda j: (0, j, 0)),
            pl.BlockSpec((HOPS, HID), lambda j: (0, 0)),
            pl.BlockSpec((HOPS, HID, OUT), lambda j: (0, 0, 0)),
            pl.BlockSpec((HOPS, OUT), lambda j: (0, 0)),
            pl.BlockSpec((HEADS, HID), lambda j: (0, 0)),
        ],
        out_specs=pl.BlockSpec((RB, OUT), lambda j: (j, 0)),
        out_shape=jax.ShapeDtypeStruct((N, OUT), jnp.float32),
    )(h3, asrc3, adst3, b3, accp, denp, bg, Wd, bd, S)


def kernel(x, edge_index_k_hops, Wg, att_src, att_dst, bg, Wd, bd):
    eye8 = jnp.eye(HEADS, dtype=jnp.float32)
    Asrc = (att_src[:, :, :, None] * eye8[None, :, None, :]).reshape(HOPS, HID, HEADS)
    Adst = (att_dst[:, :, :, None] * eye8[None, :, None, :]).reshape(HOPS, HID, HEADS)
    S = jnp.repeat(eye8, HD, axis=1)

    h3, asrc3, adst3, mxs, mxd = _prep(x, Wg, Asrc, Adst)

    b3 = _leaky(mxs[:, 0, :] + mxd[:, 0, :], 0.2)
    barr = jnp.concatenate([b3, b3], axis=1).reshape(HOPS * 16)
    pad = jnp.zeros((HOPS, N, HID - 16), jnp.float32)
    alphaA = jnp.concatenate([asrc3, adst3, pad], -1).reshape(HOPS * N, HID)
    h_flat = h3.reshape(HOPS * N, HID)
    edges_flat = edge_index_k_hops.reshape(HOPS * 2 * E)

    accp, exflat = _sc_edges(edges_flat, h_flat, alphaA, barr)
    denp, = _sc_den(edges_flat, exflat)

    return _decode(h3, asrc3, adst3, b3, accp, denp, bg, Wd, bd, S)


# speedup vs baseline: 52.5856x; 1.2737x over previous
import functools

import jax
import jax.numpy as jnp
from jax import lax
from jax.experimental import pallas as pl
from jax.experimental.pallas import tpu as pltpu
from jax.experimental.pallas import tpu_sc as plsc

N = 10000
E = 320000
HOPS = 3
HEADS = 8
HID = 128
HD = HID // HEADS
OUT = 128
DECAY = (0.6, 0.3, 0.1)

NWORKERS = 32
PER_TILE = E // NWORKERS
K = 80
NCHUNK = PER_TILE // K
ROWS_A = 640
ROWS_B = N - 15 * ROWS_A
RB = 1000
NB = N // RB


def _leaky(v, slope):
    return jnp.where(v > 0, v, v * slope)


def _prep_body(x_ref, wg_ref, asrcm_ref, adstm_ref,
               h_ref, as_ref, ad_ref, mxs_ref, mxd_ref):
    j = pl.program_id(1)
    h = jnp.dot(x_ref[...], wg_ref[0], preferred_element_type=jnp.float32)
    h_ref[0] = h
    asv = jnp.dot(h, asrcm_ref[0], preferred_element_type=jnp.float32)
    adv = jnp.dot(h, adstm_ref[0], preferred_element_type=jnp.float32)
    as_ref[0] = asv
    ad_ref[0] = adv
    bs = jnp.max(asv, axis=0)[None, None, :]
    bd = jnp.max(adv, axis=0)[None, None, :]

    @pl.when(j == 0)
    def _():
        mxs_ref[...] = bs
        mxd_ref[...] = bd

    @pl.when(j > 0)
    def _():
        mxs_ref[...] = jnp.maximum(mxs_ref[...], bs)
        mxd_ref[...] = jnp.maximum(mxd_ref[...], bd)


def _prep(x, Wg, Asrc, Adst):
    return pl.pallas_call(
        _prep_body,
        grid=(HOPS, NB),
        in_specs=[
            pl.BlockSpec((RB, HID), lambda k, j: (j, 0)),
            pl.BlockSpec((1, HID, HID), lambda k, j: (k, 0, 0)),
            pl.BlockSpec((1, HID, HEADS), lambda k, j: (k, 0, 0)),
            pl.BlockSpec((1, HID, HEADS), lambda k, j: (k, 0, 0)),
        ],
        out_specs=[
            pl.BlockSpec((1, RB, HID), lambda k, j: (k, j, 0)),
            pl.BlockSpec((1, RB, HEADS), lambda k, j: (k, j, 0)),
            pl.BlockSpec((1, RB, HEADS), lambda k, j: (k, j, 0)),
            pl.BlockSpec((1, 1, HEADS), lambda k, j: (k, 0, 0)),
            pl.BlockSpec((1, 1, HEADS), lambda k, j: (k, 0, 0)),
        ],
        out_shape=[
            jax.ShapeDtypeStruct((HOPS, N, HID), jnp.float32),
            jax.ShapeDtypeStruct((HOPS, N, HEADS), jnp.float32),
            jax.ShapeDtypeStruct((HOPS, N, HEADS), jnp.float32),
            jax.ShapeDtypeStruct((HOPS, 1, HEADS), jnp.float32),
            jax.ShapeDtypeStruct((HOPS, 1, HEADS), jnp.float32),
        ],
    )(x, Wg, Asrc, Adst)


def _sc_edges(edges, h_flat, alphaA, barr):
    mesh = plsc.VectorSubcoreMesh(core_axis_name="c", subcore_axis_name="s")

    @functools.partial(
        pl.kernel,
        mesh=mesh,
        out_type=[
            jax.ShapeDtypeStruct((HOPS * 2, N, HID), jnp.float32),
            jax.ShapeDtypeStruct((HOPS * E * 16,), jnp.float32),
        ],
        scratch_types=[
            pltpu.VMEM((K,), jnp.int32),
            pltpu.VMEM((K,), jnp.int32),
            pltpu.VMEM((K,), jnp.int32),
            pltpu.VMEM((K, HID), jnp.float32),
            pltpu.VMEM((K, HID), jnp.float32),
            pltpu.VMEM((K, 16), jnp.float32),
            pltpu.VMEM((K * 16,), jnp.float32),
            pltpu.VMEM((K, HID), jnp.float32),
            pltpu.VMEM((16,), jnp.float32),
            pltpu.VMEM_SHARED((N, HID), jnp.float32),
            pltpu.SemaphoreType.DMA,
            pltpu.SemaphoreType.DMA,
            pltpu.SemaphoreType.DMA,
        ],
    )
    def body(edges_hbm, h_hbm, aA_hbm, b_hbm,
             accp_hbm, ex_hbm,
             srcb, dstb, dstk, gdbuf, gsbuf, exbuf, packed, rows, bbuf,
             acc_sh, sem1, sem2, sem3):
        c = lax.axis_index("c")
        s = lax.axis_index("s")
        wid = s * 2 + c
        zero16 = jnp.zeros((16,), jnp.float32)

        for k in range(HOPS):
            kc = 2 * k + c

            def zrow(e, _):
                for j in range(HEADS):
                    rows[e, pl.ds(16 * j, 16)] = zero16
                exbuf[e, :] = zero16
                return 0

            lax.fori_loop(0, K, zrow, 0)

            @pl.when(s < 15)
            def _():
                for i in range(ROWS_A // K):
                    pltpu.sync_copy(rows, acc_sh.at[pl.ds(s * ROWS_A + K * i, K)])

            @pl.when(s == 15)
            def _():
                for i in range(ROWS_B // K):
                    pltpu.sync_copy(rows, acc_sh.at[pl.ds(15 * ROWS_A + K * i, K)])

            pltpu.sync_copy(b_hbm.at[pl.ds(16 * k, 16)], bbuf)
            plsc.subcore_barrier()

            def chunk(cc, _):
                base = wid * PER_TILE + cc * K
                soff = k * 2 * E + base
                pltpu.sync_copy(edges_hbm.at[pl.ds(soff, K)], srcb)
                pltpu.sync_copy(edges_hbm.at[pl.ds(soff + E, K)], dstb)
                for i in range(K // 16):
                    sl = pl.ds(16 * i, 16)
                    srcb[sl] = srcb[sl] + k * N
                    dstk[sl] = dstb[sl] + k * N
                cp_h = pltpu.async_copy(h_hbm.at[srcb], rows, sem3)
                cp_d = pltpu.async_copy(aA_hbm.at[dstk], gdbuf, sem1)
                cp_s = pltpu.async_copy(aA_hbm.at[srcb], gsbuf, sem2)
                rot = ((lax.iota(jnp.int32, 16) + 8) & 15)[:, None]
                dnums = lax.GatherDimensionNumbers(
                    offset_dims=(), collapsed_slice_dims=(0,),
                    start_index_map=(0,))
                cp_d.wait()
                cp_s.wait()
                bv = bbuf[:]

                def exloop(e, _):
                    gd = lax.gather(
                        gdbuf[e, pl.ds(0, 16)], rot, dimension_numbers=dnums,
                        slice_sizes=(1,),
                        mode=lax.GatherScatterMode.PROMISE_IN_BOUNDS)
                    sv = gsbuf[e, pl.ds(0, 16)] + gd
                    exbuf[e, :] = jnp.exp(_leaky(sv, 0.2) - bv)
                    return 0

                lax.fori_loop(0, K, exloop, 0)
                cp_h.wait()

                def sloop(e, _):
                    ev = exbuf[e, :]
                    packed[pl.ds(16 * e, 16)] = ev
                    for j in range(HEADS):
                        sl = pl.ds(16 * j, 16)
                        rows[e, sl] = rows[e, sl] * ev[j]
                    return 0

                lax.fori_loop(0, K, sloop, 0)
                pltpu.sync_copy(rows, acc_sh.at[dstb], add=True)
                pltpu.sync_copy(packed,
                                ex_hbm.at[pl.ds((k * E + base) * 16, K * 16)])
                return 0

            lax.fori_loop(0, NCHUNK, chunk, 0)
            plsc.subcore_barrier()

            @pl.when(s < 15)
            def _():
                pltpu.sync_copy(acc_sh.at[pl.ds(s * ROWS_A, ROWS_A)],
                                accp_hbm.at[kc, pl.ds(s * ROWS_A, ROWS_A)])

            @pl.when(s == 15)
            def _():
                pltpu.sync_copy(acc_sh.at[pl.ds(15 * ROWS_A, ROWS_B)],
                                accp_hbm.at[kc, pl.ds(15 * ROWS_A, ROWS_B)])

            plsc.subcore_barrier()

    return body(edges, h_flat, alphaA, barr)


def _sc_den(edges, exflat):
    mesh = plsc.VectorSubcoreMesh(core_axis_name="c", subcore_axis_name="s")

    @functools.partial(
        pl.kernel,
        mesh=mesh,
        out_type=[
            jax.ShapeDtypeStruct((HOPS * 2, N, HID), jnp.float32),
        ],
        scratch_types=[
            pltpu.VMEM((K,), jnp.int32),
            pltpu.VMEM((K * 16,), jnp.float32),
            pltpu.VMEM((K, HID), jnp.float32),
            pltpu.VMEM_SHARED((N, HID), jnp.float32),
        ],
    )
    def body(edges_hbm, ex_hbm, denp_hbm,
             dstb, packed, rows, den_sh):
        c = lax.axis_index("c")
        s = lax.axis_index("s")
        wid = s * 2 + c
        zero16 = jnp.zeros((16,), jnp.float32)

        for k in range(HOPS):
            kc = 2 * k + c

            def zrow(e, _):
                for j in range(HEADS):
                    rows[e, pl.ds(16 * j, 16)] = zero16
                return 0

            lax.fori_loop(0, K, zrow, 0)

            @pl.when(s < 15)
            def _():
                for i in range(ROWS_A // K):
                    pltpu.sync_copy(rows, den_sh.at[pl.ds(s * ROWS_A + K * i, K)])

            @pl.when(s == 15)
            def _():
                for i in range(ROWS_B // K):
                    pltpu.sync_copy(rows, den_sh.at[pl.ds(15 * ROWS_A + K * i, K)])

            plsc.subcore_barrier()

            def chunk(cc, _):
                base = wid * PER_TILE + cc * K
                soff = k * 2 * E + base
                pltpu.sync_copy(edges_hbm.at[pl.ds(soff + E, K)], dstb)
                pltpu.sync_copy(ex_hbm.at[pl.ds((k * E + base) * 16, K * 16)],
                                packed)

                def uloop(e, _):
                    rows[e, pl.ds(0, 16)] = packed[pl.ds(16 * e, 16)]
                    return 0

                lax.fori_loop(0, K, uloop, 0)
                pltpu.sync_copy(rows, den_sh.at[dstb], add=True)
                return 0

            lax.fori_loop(0, NCHUNK, chunk, 0)
            plsc.subcore_barrier()

            @pl.when(s < 15)
            def _():
                pltpu.sync_copy(den_sh.at[pl.ds(s * ROWS_A, ROWS_A)],
                                denp_hbm.at[kc, pl.ds(s * ROWS_A, ROWS_A)])

            @pl.when(s == 15)
            def _():
                pltpu.sync_copy(den_sh.at[pl.ds(15 * ROWS_A, ROWS_B)],
                                denp_hbm.at[kc, pl.ds(15 * ROWS_A, ROWS_B)])

            plsc.subcore_barrier()

    return body(edges, exflat)


def _decode_body(h_ref, as_ref, ad_ref, b_ref, accp_ref, denp_ref,
                 bg_ref, wd_ref, bd_ref, s_ref, out_ref):
    acc = jnp.zeros((RB, OUT), jnp.float32)
    S = s_ref[...]
    for k in range(HOPS):
        exs = jnp.exp(_leaky(as_ref[k] + ad_ref[k], 0.2) - b_ref[k][None, :])
        den = denp_ref[2 * k][:, :HEADS] + denp_ref[2 * k + 1][:, :HEADS] + exs
        exs128 = jnp.dot(exs, S, preferred_element_type=jnp.float32)
        rcp128 = jnp.dot(1.0 / den, S, preferred_element_type=jnp.float32)
        num = accp_ref[2 * k] + accp_ref[2 * k + 1] + h_ref[k] * exs128
        gat = num * rcp128 + bg_ref[k][None, :]
        xk = jnp.dot(gat, wd_ref[k], preferred_element_type=jnp.float32)
        xk = _leaky(xk + bd_ref[k][None, :], 0.01)
        acc = acc + DECAY[k] * xk
    out_ref[...] = acc


def _decode(h3, asrc3, adst3, b3, accp, denp, bg, Wd, bd, S):
    return pl.pallas_call(
        _decode_body,
        grid=(NB,),
        in_specs=[
            pl.BlockSpec((HOPS, RB, HID), lambda j: (0, j, 0)),
            pl.BlockSpec((HOPS, RB, HEADS), lambda j: (0, j, 0)),
            pl.BlockSpec((HOPS, RB, HEADS), lambda j: (0, j, 0)),
            pl.BlockSpec((HOPS, HEADS), lambda j: (0, 0)),
            pl.BlockSpec((HOPS * 2, RB, HID), lambda j: (0, j, 0)),
            pl.BlockSpec((HOPS * 2, RB, HID), lambda j: (0, j, 0)),
            pl.BlockSpec((HOPS, HID), lambda j: (0, 0)),
            pl.BlockSpec((HOPS, HID, OUT), lambda j: (0, 0, 0)),
            pl.BlockSpec((HOPS, OUT), lambda j: (0, 0)),
            pl.BlockSpec((HEADS, HID), lambda j: (0, 0)),
        ],
        out_specs=pl.BlockSpec((RB, OUT), lambda j: (j, 0)),
        out_shape=jax.ShapeDtypeStruct((N, OUT), jnp.float32),
    )(h3, asrc3, adst3, b3, accp, denp, bg, Wd, bd, S)


def kernel(x, edge_index_k_hops, Wg, att_src, att_dst, bg, Wd, bd):
    eye8 = jnp.eye(HEADS, dtype=jnp.float32)
    Asrc = (att_src[:, :, :, None] * eye8[None, :, None, :]).reshape(HOPS, HID, HEADS)
    Adst = (att_dst[:, :, :, None] * eye8[None, :, None, :]).reshape(HOPS, HID, HEADS)
    S = jnp.repeat(eye8, HD, axis=1)

    h3, asrc3, adst3, mxs, mxd = _prep(x, Wg, Asrc, Adst)

    b3 = _leaky(mxs[:, 0, :] + mxd[:, 0, :], 0.2)
    barr = jnp.concatenate([b3, b3], axis=1).reshape(HOPS * 16)
    pad = jnp.zeros((HOPS, N, HID - 16), jnp.float32)
    alphaA = jnp.concatenate([asrc3, adst3, pad], -1).reshape(HOPS * N, HID)
    h_flat = h3.reshape(HOPS * N, HID)
    edges_flat = edge_index_k_hops.reshape(HOPS * 2 * E)

    accp, exflat = _sc_edges(edges_flat, h_flat, alphaA, barr)
    denp, = _sc_den(edges_flat, exflat)

    return _decode(h3, asrc3, adst3, b3, accp, denp, bg, Wd, bd, S)
